# Initial kernel scaffold; baseline (speedup 1.0000x reference)
import jax
import jax.numpy as jnp
from jax.experimental import pallas as pl


def kernel(x, position_weight, value_weight, classify_weight):
    raise NotImplementedError("write your pallas kernel here")


# speedup vs baseline: 5.3401x; 5.3401x over previous
import jax
import jax.numpy as jnp
from jax.experimental import pallas as pl

D = 10000
L = 1000
P = 784
B = 4
NCLS = 10
TILE = 1024
GRID = (D + TILE - 1) // TILE


def _bundle_body(x_ref, v_ref, pos_ref, out_ref):
    v = v_ref[...]
    b1 = v[L - 1:L, :]
    t = jnp.sum((v != b1).astype(jnp.int32), axis=0, keepdims=True)
    pos = pos_ref[...]
    s = jnp.sum(pos, axis=0, keepdims=True)
    xf = x_ref[...]
    idx = jnp.clip(jnp.round(xf * (L - 1)), 0, L - 1).astype(jnp.int32)
    rows = []
    for b in range(B):
        ib = idx[b, :].reshape(P, 1)
        s1 = jnp.sum(jnp.where(ib >= t, pos, 0.0), axis=0, keepdims=True)
        rows.append(b1 * (2.0 * s1 - s))
    out_ref[...] = jnp.concatenate(rows, axis=0)


def _classify_body(bun_ref, cw_ref, out_ref):
    enc = jnp.where(bun_ref[...] > 0, 1.0, -1.0)
    out_ref[...] = jax.lax.dot_general(
        enc, cw_ref[...], (((1,), (1,)), ((), ())),
        preferred_element_type=jnp.float32)


def kernel(x, position_weight, value_weight, classify_weight):
    flat = x.reshape(B, P)
    bundled = pl.pallas_call(
        _bundle_body,
        grid=(GRID,),
        in_specs=[
            pl.BlockSpec((B, P), lambda i: (0, 0)),
            pl.BlockSpec((L, TILE), lambda i: (0, i)),
            pl.BlockSpec((P, TILE), lambda i: (0, i)),
        ],
        out_specs=pl.BlockSpec((B, TILE), lambda i: (0, i)),
        out_shape=jax.ShapeDtypeStruct((B, D), jnp.float32),
    )(flat, value_weight, position_weight)
    logit = pl.pallas_call(
        _classify_body,
        in_specs=[
            pl.BlockSpec((B, D), lambda: (0, 0)),
            pl.BlockSpec((NCLS, D), lambda: (0, 0)),
        ],
        out_specs=pl.BlockSpec((B, NCLS), lambda: (0, 0)),
        out_shape=jax.ShapeDtypeStruct((B, NCLS), jnp.float32),
    )(bundled, classify_weight)
    return logit
